# baseline (speedup 1.0000x reference)
import math

import jax
import jax.numpy as jnp
from jax import lax
from jax.experimental import pallas as pl
from jax.experimental.pallas import tpu as pltpu
from jax.experimental.pallas import tpu_sc as plsc

S = 30.0
M = 0.5
COS_M = math.cos(M)
SIN_M = math.sin(M)
TH = math.cos(math.pi - M)
MM = math.sin(math.pi - M) * M

B, C = 1024, 100000
NC, NS = 2, 16
NW = NC * NS
RPW = B // NW
CW = 768
NFULL = 130
TOFF = NFULL * CW
TW = C - TOFF
NBUF = 2


def _phi_scaled(tv):
    x = jnp.maximum(1.0 - tv * tv, 1e-30)
    i = plsc.bitcast(x, jnp.int32)
    y = plsc.bitcast(jnp.int32(0x5F3759DF) - (i >> 1), jnp.float32)
    for _ in range(3):
        y = y * (1.5 - 0.5 * x * y * y)
    sine = x * y
    phi_s = tv * (S * COS_M) - sine * (S * SIN_M)
    return jnp.where(tv > TH, phi_s, tv * S - S * MM)


def _patch(bin_ref, bout_ref, lab_v, base, c0, w):
    lanes = lax.iota(jnp.int32, 16)
    for half in range(2):
        rvec = lanes + half * 16
        labv = plsc.load_gather(lab_v, [base + rvec])
        loc = labv - c0
        inb = (loc >= 0) & (loc < w)
        li = jnp.clip(loc, 0, w - 1)
        tv = plsc.load_gather(bin_ref, [rvec, li])
        phi_s = _phi_scaled(tv)
        plsc.store_scatter(bout_ref, [rvec, li], phi_s, mask=inb)


def _sc_body(cos_hbm, lab_hbm, out_hbm, lab_v, bin_v, bout_v, tin_v, tout_v, sin, sout):
    cid = lax.axis_index("c")
    sid = lax.axis_index("s")
    wid = sid * NC + cid
    base = wid * RPW
    rows = pl.ds(base, RPW)

    pltpu.sync_copy(lab_hbm, lab_v)

    for b in range(NBUF):
        pltpu.async_copy(cos_hbm.at[rows, pl.ds(b * CW, CW)], bin_v.at[b], sin.at[b])

    def scale_rows(ref_in, ref_out, nvec):
        def row(r, c):
            for u in range(nvec):
                ref_out[r, pl.ds(u * 16, 16)] = ref_in[r, pl.ds(u * 16, 16)] * S
            return c

        lax.fori_loop(0, RPW, row, 0, unroll=False)

    def outer(g, carry):
        for b in range(NBUF):
            t = g * NBUF + b
            c0 = t * CW
            pltpu.make_async_copy(
                cos_hbm.at[rows, pl.ds(c0, CW)], bin_v.at[b], sin.at[b]
            ).wait()

            @pl.when(t >= NBUF)
            def _():
                pc0 = (t - NBUF) * CW
                pltpu.make_async_copy(
                    bout_v.at[b], out_hbm.at[rows, pl.ds(pc0, CW)], sout.at[b]
                ).wait()

            scale_rows(bin_v.at[b], bout_v.at[b], CW // 16)
            _patch(bin_v.at[b], bout_v.at[b], lab_v, base, c0, CW)

            pltpu.async_copy(bout_v.at[b], out_hbm.at[rows, pl.ds(c0, CW)], sout.at[b])

            @pl.when(t + NBUF < NFULL)
            def _():
                nc0 = (t + NBUF) * CW
                pltpu.async_copy(
                    cos_hbm.at[rows, pl.ds(nc0, CW)], bin_v.at[b], sin.at[b]
                )
        return carry

    lax.fori_loop(0, NFULL // NBUF, outer, 0, unroll=False)

    for b in range(NBUF):
        c0 = (NFULL - NBUF + b) * CW
        pltpu.make_async_copy(
            bout_v.at[b], out_hbm.at[rows, pl.ds(c0, CW)], sout.at[b]
        ).wait()

    pltpu.sync_copy(cos_hbm.at[rows, pl.ds(TOFF, TW)], tin_v)
    scale_rows(tin_v, tout_v, TW // 16)
    _patch(tin_v, tout_v, lab_v, base, TOFF, TW)
    pltpu.sync_copy(tout_v, out_hbm.at[rows, pl.ds(TOFF, TW)])


_sc_call = pl.kernel(
    _sc_body,
    out_type=jax.ShapeDtypeStruct((B, C), jnp.float32),
    mesh=plsc.VectorSubcoreMesh(core_axis_name="c", subcore_axis_name="s"),
    compiler_params=pltpu.CompilerParams(needs_layout_passes=False),
    scratch_types=[
        pltpu.VMEM((B,), jnp.int32),
        pltpu.VMEM((NBUF, RPW, CW), jnp.float32),
        pltpu.VMEM((NBUF, RPW, CW), jnp.float32),
        pltpu.VMEM((RPW, TW), jnp.float32),
        pltpu.VMEM((RPW, TW), jnp.float32),
        pltpu.SemaphoreType.DMA((NBUF,)),
        pltpu.SemaphoreType.DMA((NBUF,)),
    ],
)


def kernel(cosine, label):
    return _sc_call(cosine, label.astype(jnp.int32))


# speedup vs baseline: 1.0024x; 1.0024x over previous
import math

import jax
import jax.numpy as jnp
from jax import lax
from jax.experimental import pallas as pl
from jax.experimental.pallas import tpu as pltpu
from jax.experimental.pallas import tpu_sc as plsc

S = 30.0
M = 0.5
COS_M = math.cos(M)
SIN_M = math.sin(M)
TH = math.cos(math.pi - M)
MM = math.sin(math.pi - M) * M

B, C = 1024, 100000
NC, NS = 2, 16
NW = NC * NS
RPW = B // NW
CW = 512
NFULL = 195
TOFF = NFULL * CW
TW = C - TOFF
NBUF = 3


def _phi_scaled(tv):
    x = jnp.maximum(1.0 - tv * tv, 1e-30)
    i = plsc.bitcast(x, jnp.int32)
    y = plsc.bitcast(jnp.int32(0x5F3759DF) - (i >> 1), jnp.float32)
    for _ in range(3):
        y = y * (1.5 - 0.5 * x * y * y)
    sine = x * y
    phi_s = tv * (S * COS_M) - sine * (S * SIN_M)
    return jnp.where(tv > TH, phi_s, tv * S - S * MM)


def _patch(bin_ref, bout_ref, lab_v, base, c0, w):
    lanes = lax.iota(jnp.int32, 16)
    for half in range(2):
        rvec = lanes + half * 16
        labv = plsc.load_gather(lab_v, [base + rvec])
        loc = labv - c0
        inb = (loc >= 0) & (loc < w)
        li = jnp.clip(loc, 0, w - 1)
        tv = plsc.load_gather(bin_ref, [rvec, li])
        phi_s = _phi_scaled(tv)
        plsc.store_scatter(bout_ref, [rvec, li], phi_s, mask=inb)


def _sc_body(cos_hbm, lab_hbm, out_hbm, lab_v, bin_v, bout_v, tin_v, tout_v, sin, sout):
    cid = lax.axis_index("c")
    sid = lax.axis_index("s")
    wid = sid * NC + cid
    base = wid * RPW
    rows = pl.ds(base, RPW)

    pltpu.sync_copy(lab_hbm, lab_v)

    for b in range(NBUF):
        pltpu.async_copy(cos_hbm.at[rows, pl.ds(b * CW, CW)], bin_v.at[b], sin.at[b])

    def scale_rows(ref_in, ref_out, nvec):
        def row(r, c):
            for u in range(nvec):
                ref_out[r, pl.ds(u * 16, 16)] = ref_in[r, pl.ds(u * 16, 16)] * S
            return c

        lax.fori_loop(0, RPW, row, 0, unroll=False)

    def outer(g, carry):
        for b in range(NBUF):
            t = g * NBUF + b
            c0 = t * CW
            pltpu.make_async_copy(
                cos_hbm.at[rows, pl.ds(c0, CW)], bin_v.at[b], sin.at[b]
            ).wait()

            @pl.when(t >= NBUF)
            def _():
                pc0 = (t - NBUF) * CW
                pltpu.make_async_copy(
                    bout_v.at[b], out_hbm.at[rows, pl.ds(pc0, CW)], sout.at[b]
                ).wait()

            scale_rows(bin_v.at[b], bout_v.at[b], CW // 16)
            _patch(bin_v.at[b], bout_v.at[b], lab_v, base, c0, CW)

            pltpu.async_copy(bout_v.at[b], out_hbm.at[rows, pl.ds(c0, CW)], sout.at[b])

            @pl.when(t + NBUF < NFULL)
            def _():
                nc0 = (t + NBUF) * CW
                pltpu.async_copy(
                    cos_hbm.at[rows, pl.ds(nc0, CW)], bin_v.at[b], sin.at[b]
                )
        return carry

    lax.fori_loop(0, NFULL // NBUF, outer, 0, unroll=False)

    for b in range(NBUF):
        c0 = (NFULL - NBUF + b) * CW
        pltpu.make_async_copy(
            bout_v.at[b], out_hbm.at[rows, pl.ds(c0, CW)], sout.at[b]
        ).wait()

    pltpu.sync_copy(cos_hbm.at[rows, pl.ds(TOFF, TW)], tin_v)
    scale_rows(tin_v, tout_v, TW // 16)
    _patch(tin_v, tout_v, lab_v, base, TOFF, TW)
    pltpu.sync_copy(tout_v, out_hbm.at[rows, pl.ds(TOFF, TW)])


_sc_call = pl.kernel(
    _sc_body,
    out_type=jax.ShapeDtypeStruct((B, C), jnp.float32),
    mesh=plsc.VectorSubcoreMesh(core_axis_name="c", subcore_axis_name="s"),
    compiler_params=pltpu.CompilerParams(needs_layout_passes=False),
    scratch_types=[
        pltpu.VMEM((B,), jnp.int32),
        pltpu.VMEM((NBUF, RPW, CW), jnp.float32),
        pltpu.VMEM((NBUF, RPW, CW), jnp.float32),
        pltpu.VMEM((RPW, TW), jnp.float32),
        pltpu.VMEM((RPW, TW), jnp.float32),
        pltpu.SemaphoreType.DMA((NBUF,)),
        pltpu.SemaphoreType.DMA((NBUF,)),
    ],
)


def kernel(cosine, label):
    return _sc_call(cosine, label.astype(jnp.int32))


# speedup vs baseline: 1.0048x; 1.0025x over previous
import math

import jax
import jax.numpy as jnp
from jax import lax
from jax.experimental import pallas as pl
from jax.experimental.pallas import tpu as pltpu
from jax.experimental.pallas import tpu_sc as plsc

S = 30.0
M = 0.5
COS_M = math.cos(M)
SIN_M = math.sin(M)
TH = math.cos(math.pi - M)
MM = math.sin(math.pi - M) * M

B, C = 1024, 100000
NC, NS = 2, 16
NW = NC * NS
RPW = B // NW
CW = 512
NFULL = 195
TOFF = NFULL * CW
TW = C - TOFF
NBUF = 3


def _phi_scaled(tv):
    x = jnp.maximum(1.0 - tv * tv, 1e-30)
    i = plsc.bitcast(x, jnp.int32)
    y = plsc.bitcast(jnp.int32(0x5F3759DF) - (i >> 1), jnp.float32)
    for _ in range(3):
        y = y * (1.5 - 0.5 * x * y * y)
    sine = x * y
    phi_s = tv * (S * COS_M) - sine * (S * SIN_M)
    return jnp.where(tv > TH, phi_s, tv * S - S * MM)


def _patch(bin_ref, bout_ref, lab_v, base, c0, w):
    lanes = lax.iota(jnp.int32, 16)
    for half in range(2):
        rvec = lanes + half * 16
        labv = plsc.load_gather(lab_v, [base + rvec])
        loc = labv - c0
        inb = (loc >= 0) & (loc < w)
        li = jnp.clip(loc, 0, w - 1)
        tv = plsc.load_gather(bin_ref, [rvec, li])
        phi_s = _phi_scaled(tv)
        plsc.store_scatter(bout_ref, [rvec, li], phi_s, mask=inb)


def _sc_body(cos_hbm, lab_hbm, out_hbm, lab_v, bin_v, bout_v, tin_v, tout_v, sin, sout):
    cid = lax.axis_index("c")
    sid = lax.axis_index("s")
    wid = sid * NC + cid
    base = wid * RPW
    rows = pl.ds(base, RPW)

    pltpu.sync_copy(lab_hbm, lab_v)

    for b in range(NBUF):
        pltpu.async_copy(cos_hbm.at[rows, pl.ds(b * CW, CW)], bin_v.at[b], sin.at[b])

    def scale_rows(ref_in, ref_out, nvec):
        def row(r, c):
            for u in range(nvec):
                ref_out[r, pl.ds(u * 16, 16)] = ref_in[r, pl.ds(u * 16, 16)] * S
            return c

        lax.fori_loop(0, RPW, row, 0, unroll=False)

    def outer(g, carry):
        for b in range(NBUF):
            t = g * NBUF + b
            c0 = t * CW
            pltpu.make_async_copy(
                cos_hbm.at[rows, pl.ds(c0, CW)], bin_v.at[b], sin.at[b]
            ).wait()

            @pl.when(t >= NBUF)
            def _():
                pc0 = (t - NBUF) * CW
                pltpu.make_async_copy(
                    bout_v.at[b], out_hbm.at[rows, pl.ds(pc0, CW)], sout.at[b]
                ).wait()

            scale_rows(bin_v.at[b], bout_v.at[b], CW // 16)
            _patch(bin_v.at[b], bout_v.at[b], lab_v, base, c0, CW)

            pltpu.async_copy(bout_v.at[b], out_hbm.at[rows, pl.ds(c0, CW)], sout.at[b])

            @pl.when(t + NBUF < NFULL)
            def _():
                nc0 = (t + NBUF) * CW
                pltpu.async_copy(
                    cos_hbm.at[rows, pl.ds(nc0, CW)], bin_v.at[b], sin.at[b]
                )
        return carry

    lax.fori_loop(0, NFULL // NBUF, outer, 0, unroll=False)

    for b in range(NBUF):
        c0 = (NFULL - NBUF + b) * CW
        pltpu.make_async_copy(
            bout_v.at[b], out_hbm.at[rows, pl.ds(c0, CW)], sout.at[b]
        ).wait()

    pltpu.sync_copy(cos_hbm.at[rows, pl.ds(TOFF, TW)], tin_v)
    scale_rows(tin_v, tout_v, TW // 16)
    _patch(tin_v, tout_v, lab_v, base, TOFF, TW)
    pltpu.sync_copy(tout_v, out_hbm.at[rows, pl.ds(TOFF, TW)])


_sc_call = pl.kernel(
    _sc_body,
    out_type=jax.ShapeDtypeStruct((B, C), jnp.float32),
    mesh=plsc.VectorSubcoreMesh(core_axis_name="c", subcore_axis_name="s"),
    compiler_params=pltpu.CompilerParams(
        needs_layout_passes=False, use_tc_tiling_on_sc=True
    ),
    scratch_types=[
        pltpu.VMEM((B,), jnp.int32),
        pltpu.VMEM((NBUF, RPW, CW), jnp.float32),
        pltpu.VMEM((NBUF, RPW, CW), jnp.float32),
        pltpu.VMEM((RPW, TW), jnp.float32),
        pltpu.VMEM((RPW, TW), jnp.float32),
        pltpu.SemaphoreType.DMA((NBUF,)),
        pltpu.SemaphoreType.DMA((NBUF,)),
    ],
)


def kernel(cosine, label):
    return _sc_call(cosine, label.astype(jnp.int32))


# speedup vs baseline: 3.2900x; 3.2742x over previous
import math

import jax
import jax.numpy as jnp
from jax import lax
from jax.experimental import pallas as pl
from jax.experimental.pallas import tpu as pltpu
from jax.experimental.pallas import tpu_sc as plsc

S = 30.0
M = 0.5
COS_M = math.cos(M)
SIN_M = math.sin(M)
TH = math.cos(math.pi - M)
MM = math.sin(math.pi - M) * M

B, C = 1024, 100000
NC, NS = 2, 16
NW = NC * NS
RPW = 3120
GR = 24
NCH = RPW // GR
NBUF = 2
EPI0 = NW * RPW
EPW = (C - EPI0) // 8


def _phi_scaled(tv):
    x = jnp.maximum(1.0 - tv * tv, 1e-30)
    i = plsc.bitcast(x, jnp.int32)
    y = plsc.bitcast(jnp.int32(0x5F3759DF) - (i >> 1), jnp.float32)
    for _ in range(3):
        y = y * (1.5 - 0.5 * x * y * y)
    sine = x * y
    phi_s = tv * (S * COS_M) - sine * (S * SIN_M)
    return jnp.where(tv > TH, phi_s, tv * S - S * MM)


def _sc_body(cos_hbm, lab_hbm, out_hbm, lab_v, hit_c, hit_i, bin_v,
             bout_v, sin, sout):
    cid = lax.axis_index("c")
    sid = lax.axis_index("s")
    wid = sid * NC + cid
    r0 = wid * RPW
    er0 = EPI0 + wid * 8
    has_epi = wid < EPW

    for b in range(NBUF):
        pltpu.async_copy(cos_hbm.at[pl.ds(r0 + b * GR, GR)], bin_v.at[b],
                         sin.at[b])

    pltpu.sync_copy(lab_hbm, lab_v)

    lanes = lax.iota(jnp.int32, 16)

    def build(k, nh):
        lv = lab_v[pl.ds(k * 16, 16)]
        hit = (lv >= r0) & (lv < r0 + RPW)
        hit = hit | (has_epi & (lv >= er0) & (lv < er0 + 8))
        hi = hit.astype(jnp.int32)
        pos = nh + jnp.cumsum(hi) - hi
        plsc.store_scatter(hit_c, [pos], lv, mask=hit)
        plsc.store_scatter(hit_i, [pos], k * 16 + lanes, mask=hit)
        return nh + plsc.all_reduce_population_count(hit)

    nh_v = lax.fori_loop(0, B // 16, build, jnp.zeros((16,), jnp.int32),
                         unroll=False)
    nh = jnp.max(nh_v)

    def patch(bin_ref, bout_ref, c0, gr):
        nv = (nh + 15) // 16

        def pv(k, c):
            li = k * 16 + lanes
            vm = li < nh
            lc = plsc.load_gather(hit_c, [li])
            iv = plsc.load_gather(hit_i, [li])
            loc = lc - c0
            inb = vm & (loc >= 0) & (loc < gr)
            lr = jnp.clip(loc, 0, gr - 1)
            iv = jnp.clip(iv, 0, B - 1)
            tv = plsc.load_gather(bin_ref, [lr, iv], mask=inb)
            plsc.store_scatter(bout_ref, [lr, iv], _phi_scaled(tv), mask=inb)
            return c

        lax.fori_loop(0, nv, pv, 0, unroll=False)

    def scale_rows(ref_in, ref_out, nrows):
        def row(r, c):
            for u in range(B // 16):
                ref_out[r, pl.ds(u * 16, 16)] = ref_in[r, pl.ds(u * 16, 16)] * S
            return c

        lax.fori_loop(0, nrows, row, 0, unroll=False)

    def outer(g, carry):
        for b in range(NBUF):
            t = g * NBUF + b
            c0 = r0 + t * GR
            pltpu.make_async_copy(
                cos_hbm.at[pl.ds(c0, GR)], bin_v.at[b], sin.at[b]
            ).wait()

            @pl.when(t >= NBUF)
            def _():
                pc0 = r0 + (t - NBUF) * GR
                pltpu.make_async_copy(
                    bout_v.at[b], out_hbm.at[pl.ds(pc0, GR)], sout.at[b]
                ).wait()

            scale_rows(bin_v.at[b], bout_v.at[b], GR)
            patch(bin_v.at[b], bout_v.at[b], c0, GR)

            pltpu.async_copy(bout_v.at[b], out_hbm.at[pl.ds(c0, GR)],
                             sout.at[b])

            @pl.when(t + NBUF < NCH)
            def _():
                nc0 = r0 + (t + NBUF) * GR
                pltpu.async_copy(cos_hbm.at[pl.ds(nc0, GR)], bin_v.at[b],
                                 sin.at[b])
        return carry

    lax.fori_loop(0, NCH // NBUF, outer, 0, unroll=False)

    for b in range(NBUF):
        c0 = r0 + (NCH - NBUF + b) * GR
        pltpu.make_async_copy(
            bout_v.at[b], out_hbm.at[pl.ds(c0, GR)], sout.at[b]
        ).wait()

    @pl.when(has_epi)
    def _():
        pltpu.sync_copy(cos_hbm.at[pl.ds(er0, 8)], bin_v.at[0, pl.ds(0, 8)])
        scale_rows(bin_v.at[0], bout_v.at[0], 8)
        patch(bin_v.at[0], bout_v.at[0], er0, 8)
        pltpu.sync_copy(bout_v.at[0, pl.ds(0, 8)], out_hbm.at[pl.ds(er0, 8)])


_sc_call = pl.kernel(
    _sc_body,
    out_type=jax.ShapeDtypeStruct((C, B), jnp.float32),
    mesh=plsc.VectorSubcoreMesh(core_axis_name="c", subcore_axis_name="s"),
    compiler_params=pltpu.CompilerParams(needs_layout_passes=False),
    scratch_types=[
        pltpu.VMEM((B,), jnp.int32),
        pltpu.VMEM((B,), jnp.int32),
        pltpu.VMEM((B,), jnp.int32),
        pltpu.VMEM((NBUF, GR, B), jnp.float32),
        pltpu.VMEM((NBUF, GR, B), jnp.float32),
        pltpu.SemaphoreType.DMA((NBUF,)),
        pltpu.SemaphoreType.DMA((NBUF,)),
    ],
)


def kernel(cosine, label):
    out_t = _sc_call(cosine.T, label.astype(jnp.int32))
    return out_t.T
